# baseline (speedup 1.0000x reference)
import functools

import jax
import jax.numpy as jnp
from jax import lax
from jax.experimental import pallas as pl
from jax.experimental.pallas import tpu as pltpu
from jax.experimental.pallas import tpu_sc as plsc

N_NODES = 10000
D_FEAT = 256
B = 4096
K = 20
KS = K + 1
LANES = 16
DH = D_FEAT // 2
NG = DH // LANES

try:
    _info = plsc.get_sparse_core_info()
    NC, NS = _info.num_cores, _info.num_subcores
except Exception:
    NC, NS = 2, 16
NW = NC * NS
B_PER_W = B // NW
BC = 16
BCK = BC * KS
NCH = B_PER_W // BC


def _combine_body(nf_ref, mem_ref, o_ref):
    s = nf_ref[...] + mem_ref[...]
    lo = lax.bitcast_convert_type(
        s[:, :DH].astype(jnp.bfloat16), jnp.uint16).astype(jnp.uint32)
    hi = lax.bitcast_convert_type(
        s[:, DH:].astype(jnp.bfloat16), jnp.uint16).astype(jnp.uint32)
    o_ref[...] = lax.bitcast_convert_type((hi << 16) | lo, jnp.int32)


def _combine(node_features, memory):
    grid = 10
    rows = N_NODES // grid
    return pl.pallas_call(
        _combine_body,
        out_shape=jax.ShapeDtypeStruct((N_NODES, DH), jnp.int32),
        grid=(grid,),
        in_specs=[
            pl.BlockSpec((rows, D_FEAT), lambda i: (i, 0)),
            pl.BlockSpec((rows, D_FEAT), lambda i: (i, 0)),
        ],
        out_specs=pl.BlockSpec((rows, DH), lambda i: (i, 0)),
    )(node_features, memory)


def _sc_body(table, idxw, out, nidx_v, nrows_v, out_v, sems, osems):
    wid = lax.axis_index("s") * NC + lax.axis_index("c")
    wbase = wid * B_PER_W

    pltpu.async_copy(idxw.at[pl.ds(wbase, B_PER_W)], nidx_v,
                     osems[0]).wait()

    def gathers(c, s):
        return [
            pltpu.make_async_copy(
                table.at[nidx_v.at[c * BC + j, pl.ds(0, KS)]],
                nrows_v[s].at[pl.ds(j * KS, KS)], sems[s])
            for j in range(BC)
        ]

    def issue(c, s):
        for cp in gathers(c, s):
            cp.start()

    def drain(c, s):
        pltpu.make_async_copy(table.at[pl.ds(0, BCK)], nrows_v[s],
                              sems[s]).wait()

    def row_pairs(ref, r, scale):
        outs = [None] * (2 * NG)
        for g in range(NG):
            words = ref[r, pl.ds(g * LANES, LANES)]
            pairs = plsc.bitcast(words, jnp.bfloat16)
            a, b = plsc.unpack(pairs, format=plsc.PackFormat.INTERLEAVED,
                               preferred_element_type=jnp.float32)
            outs[g] = scale * a
            outs[g + NG] = scale * b
        return tuple(outs)

    def out_copy(c, s):
        return pltpu.make_async_copy(
            out_v[s], out.at[pl.ds(wbase + c * BC, BC)], osems[s])

    def compute(c, s):
        @pl.when(c >= 2)
        def _():
            out_copy(c - 2, s).wait()

        def bbody(j, carry):
            brow = jnp.full((LANES,), c * BC + j, jnp.int32)

            def weight(k):
                bits = plsc.load_gather(
                    nidx_v, [brow, jnp.full((LANES,), KS + k, jnp.int32)])
                return plsc.bitcast(bits, jnp.float32) * 0.1

            accs = row_pairs(nrows_v[s], j * KS, weight(0))
            for k in range(1, KS):
                rows = row_pairs(nrows_v[s], j * KS + k, weight(k))
                accs = tuple(acc + rv for acc, rv in zip(accs, rows))
            for g in range(2 * NG):
                out_v[s][j, pl.ds(g * LANES, LANES)] = accs[g]
            return carry

        lax.fori_loop(0, BC, bbody, 0)
        out_copy(c, s).start()

    issue(0, 0)

    def pair(i, carry):
        c0 = 2 * i
        issue(c0 + 1, 1)
        drain(c0, 0)
        compute(c0, 0)

        @pl.when(c0 + 2 < NCH)
        def _():
            issue(c0 + 2, 0)

        drain(c0 + 1, 1)
        compute(c0 + 1, 1)
        return carry

    lax.fori_loop(0, NCH // 2, pair, 0)
    out_copy(NCH - 2, 0).wait()
    out_copy(NCH - 1, 1).wait()


@functools.partial(jax.jit, static_argnames=())
def _run(combined, idxw):
    mesh = plsc.VectorSubcoreMesh(core_axis_name="c", subcore_axis_name="s")
    sc_kernel = pl.kernel(
        _sc_body,
        out_type=jax.ShapeDtypeStruct((B, D_FEAT), jnp.float32),
        mesh=mesh,
        compiler_params=pltpu.CompilerParams(needs_layout_passes=False),
        scratch_types=[
            pltpu.VMEM((B_PER_W, 2 * KS), jnp.int32),
            [pltpu.VMEM((BCK, DH), jnp.int32)] * 2,
            [pltpu.VMEM((BC, D_FEAT), jnp.float32)] * 2,
            [pltpu.SemaphoreType.DMA] * 2,
            [pltpu.SemaphoreType.DMA] * 2,
        ],
    )
    return sc_kernel(combined, idxw)


def kernel(node_features, edge_features, memory, w_time, b_time, timestamps,
           edge_times, tppr_weights, source_nodes, neighbors, edge_idxs):
    combined = _combine(node_features, memory)
    idxw = jnp.concatenate(
        [neighbors.astype(jnp.int32),
         source_nodes.astype(jnp.int32)[:, None],
         lax.bitcast_convert_type(tppr_weights, jnp.int32),
         lax.bitcast_convert_type(
             jnp.full((B, 1), 9.0, jnp.float32), jnp.int32)],
        axis=1)
    return _run(combined, idxw)


# speedup vs baseline: 1.0059x; 1.0059x over previous
import functools

import jax
import jax.numpy as jnp
from jax import lax
from jax.experimental import pallas as pl
from jax.experimental.pallas import tpu as pltpu
from jax.experimental.pallas import tpu_sc as plsc

N_NODES = 10000
D_FEAT = 256
B = 4096
K = 20
LANES = 16
DH = D_FEAT // 2
NG = DH // LANES

try:
    _info = plsc.get_sparse_core_info()
    NC, NS = _info.num_cores, _info.num_subcores
except Exception:
    NC, NS = 2, 16
NW = NC * NS
B_PER_W = B // NW
BC = 16
BCK = BC * K
NCH = B_PER_W // BC


def _combine_body(nf_ref, mem_ref, o_ref):
    s = nf_ref[...] + mem_ref[...]
    lo = lax.bitcast_convert_type(
        s[:, :DH].astype(jnp.bfloat16), jnp.uint16).astype(jnp.uint32)
    hi = lax.bitcast_convert_type(
        s[:, DH:].astype(jnp.bfloat16), jnp.uint16).astype(jnp.uint32)
    o_ref[...] = lax.bitcast_convert_type((hi << 16) | lo, jnp.int32)


def _combine(node_features, memory):
    grid = 10
    rows = N_NODES // grid
    return pl.pallas_call(
        _combine_body,
        out_shape=jax.ShapeDtypeStruct((N_NODES, DH), jnp.int32),
        grid=(grid,),
        in_specs=[
            pl.BlockSpec((rows, D_FEAT), lambda i: (i, 0)),
            pl.BlockSpec((rows, D_FEAT), lambda i: (i, 0)),
        ],
        out_specs=pl.BlockSpec((rows, DH), lambda i: (i, 0)),
    )(node_features, memory)


def _sc_body(table, src_idx, idxw, out,
             nidx_v, sidx_v, nrows_v, srows_v, out_v, sems, osems):
    wid = lax.axis_index("s") * NC + lax.axis_index("c")
    wbase = wid * B_PER_W

    stage = [
        pltpu.async_copy(idxw.at[pl.ds(wbase, B_PER_W)], nidx_v, osems[0]),
        pltpu.async_copy(src_idx.at[pl.ds(wbase, B_PER_W)], sidx_v, osems[0]),
    ]
    for cp in stage:
        cp.wait()

    def gathers(c, s):
        cps = [
            pltpu.make_async_copy(
                table.at[nidx_v.at[c * BC + j, pl.ds(0, K)]],
                nrows_v[s].at[pl.ds(j * K, K)], sems[s])
            for j in range(BC)
        ]
        cps.append(pltpu.make_async_copy(
            table.at[sidx_v.at[pl.ds(c * BC, BC)]], srows_v[s], sems[s]))
        return cps

    def issue(c, s):
        for cp in gathers(c, s):
            cp.start()

    def drain(c, s):
        pltpu.make_async_copy(table.at[pl.ds(0, BCK)], nrows_v[s],
                              sems[s]).wait()
        pltpu.make_async_copy(table.at[pl.ds(0, BC)], srows_v[s],
                              sems[s]).wait()

    def row_pairs(ref, r, scale):
        outs = [None] * (2 * NG)
        for g in range(NG):
            words = ref[r, pl.ds(g * LANES, LANES)]
            pairs = plsc.bitcast(words, jnp.bfloat16)
            a, b = plsc.unpack(pairs, format=plsc.PackFormat.INTERLEAVED,
                               preferred_element_type=jnp.float32)
            outs[g] = scale * a
            outs[g + NG] = scale * b
        return tuple(outs)

    def out_copy(c, s):
        return pltpu.make_async_copy(
            out_v[s], out.at[pl.ds(wbase + c * BC, BC)], osems[s])

    def compute(c, s):
        @pl.when(c >= 2)
        def _():
            out_copy(c - 2, s).wait()

        def bbody(j, carry):
            accs = row_pairs(srows_v[s], j, 0.9)
            brow = jnp.full((LANES,), c * BC + j, jnp.int32)
            for k in range(K):
                swi = plsc.load_gather(
                    nidx_v, [brow, jnp.full((LANES,), K + k, jnp.int32)])
                sw = plsc.bitcast(swi, jnp.float32) * 0.1
                rows = row_pairs(nrows_v[s], j * K + k, sw)
                accs = tuple(acc + rv for acc, rv in zip(accs, rows))
            for g in range(2 * NG):
                out_v[s][j, pl.ds(g * LANES, LANES)] = accs[g]
            return carry

        lax.fori_loop(0, BC, bbody, 0)
        out_copy(c, s).start()

    issue(0, 0)

    def pair(i, carry):
        c0 = 2 * i
        issue(c0 + 1, 1)
        drain(c0, 0)
        compute(c0, 0)

        @pl.when(c0 + 2 < NCH)
        def _():
            issue(c0 + 2, 0)

        drain(c0 + 1, 1)
        compute(c0 + 1, 1)
        return carry

    lax.fori_loop(0, NCH // 2, pair, 0)
    out_copy(NCH - 2, 0).wait()
    out_copy(NCH - 1, 1).wait()


@functools.partial(jax.jit, static_argnames=())
def _run(combined, source_nodes, idxw):
    mesh = plsc.VectorSubcoreMesh(core_axis_name="c", subcore_axis_name="s")
    sc_kernel = pl.kernel(
        _sc_body,
        out_type=jax.ShapeDtypeStruct((B, D_FEAT), jnp.float32),
        mesh=mesh,
        compiler_params=pltpu.CompilerParams(needs_layout_passes=False),
        scratch_types=[
            pltpu.VMEM((B_PER_W, 2 * K), jnp.int32),
            pltpu.VMEM((B_PER_W,), jnp.int32),
            [pltpu.VMEM((BCK, DH), jnp.int32)] * 2,
            [pltpu.VMEM((BC, DH), jnp.int32)] * 2,
            [pltpu.VMEM((BC, D_FEAT), jnp.float32)] * 2,
            [pltpu.SemaphoreType.DMA] * 2,
            [pltpu.SemaphoreType.DMA] * 2,
        ],
    )
    return sc_kernel(combined, source_nodes, idxw)


def kernel(node_features, edge_features, memory, w_time, b_time, timestamps,
           edge_times, tppr_weights, source_nodes, neighbors, edge_idxs):
    combined = _combine(node_features, memory)
    idxw = jnp.concatenate(
        [neighbors.astype(jnp.int32),
         lax.bitcast_convert_type(tppr_weights, jnp.int32)], axis=1)
    return _run(combined, source_nodes.astype(jnp.int32), idxw)


# speedup vs baseline: 1.0399x; 1.0337x over previous
import functools

import jax
import jax.numpy as jnp
from jax import lax
from jax.experimental import pallas as pl
from jax.experimental.pallas import tpu as pltpu
from jax.experimental.pallas import tpu_sc as plsc

N_NODES = 10000
D_FEAT = 256
B = 4096
K = 20
LANES = 16
DH = D_FEAT // 2
NG = DH // LANES

try:
    _info = plsc.get_sparse_core_info()
    NC, NS = _info.num_cores, _info.num_subcores
except Exception:
    NC, NS = 2, 16
NW = NC * NS
B_PER_W = B // NW
BC = 16
BCK = BC * K
NCH = B_PER_W // BC


def _combine_body(nf_ref, mem_ref, o_ref):
    s = nf_ref[...] + mem_ref[...]
    lo = lax.bitcast_convert_type(
        s[:, :DH].astype(jnp.bfloat16), jnp.uint16).astype(jnp.uint32)
    hi = lax.bitcast_convert_type(
        s[:, DH:].astype(jnp.bfloat16), jnp.uint16).astype(jnp.uint32)
    o_ref[...] = lax.bitcast_convert_type((hi << 16) | lo, jnp.int32)


def _combine(node_features, memory):
    grid = 5
    rows = N_NODES // grid
    return pl.pallas_call(
        _combine_body,
        out_shape=jax.ShapeDtypeStruct((N_NODES, DH), jnp.int32),
        grid=(grid,),
        in_specs=[
            pl.BlockSpec((rows, D_FEAT), lambda i: (i, 0)),
            pl.BlockSpec((rows, D_FEAT), lambda i: (i, 0)),
        ],
        out_specs=pl.BlockSpec((rows, DH), lambda i: (i, 0)),
    )(node_features, memory)


def _sc_body(table, src_idx, idxw, out,
             nidx_v, sidx_v, nrows_v, srows_v, out_v, sems, osems):
    wid = lax.axis_index("s") * NC + lax.axis_index("c")
    wbase = wid * B_PER_W

    stage = [
        pltpu.async_copy(idxw.at[pl.ds(wbase, B_PER_W)], nidx_v, osems[0]),
        pltpu.async_copy(src_idx.at[pl.ds(wbase, B_PER_W)], sidx_v, osems[0]),
    ]
    for cp in stage:
        cp.wait()

    def gathers(c, s):
        cps = [
            pltpu.make_async_copy(
                table.at[nidx_v.at[c * BC + j, pl.ds(0, K)]],
                nrows_v[s].at[pl.ds(j * K, K)], sems[s])
            for j in range(BC)
        ]
        cps.append(pltpu.make_async_copy(
            table.at[sidx_v.at[pl.ds(c * BC, BC)]], srows_v[s], sems[s]))
        return cps

    def issue(c, s):
        for cp in gathers(c, s):
            cp.start()

    def drain(c, s):
        pltpu.make_async_copy(table.at[pl.ds(0, BCK)], nrows_v[s],
                              sems[s]).wait()
        pltpu.make_async_copy(table.at[pl.ds(0, BC)], srows_v[s],
                              sems[s]).wait()

    def row_pairs(ref, r, scale):
        outs = [None] * (2 * NG)
        for g in range(NG):
            words = ref[r, pl.ds(g * LANES, LANES)]
            pairs = plsc.bitcast(words, jnp.bfloat16)
            a, b = plsc.unpack(pairs, format=plsc.PackFormat.INTERLEAVED,
                               preferred_element_type=jnp.float32)
            outs[g] = scale * a
            outs[g + NG] = scale * b
        return tuple(outs)

    def out_copy(c, s):
        return pltpu.make_async_copy(
            out_v[s], out.at[pl.ds(wbase + c * BC, BC)], osems[s])

    def compute(c, s):
        @pl.when(c >= 2)
        def _():
            out_copy(c - 2, s).wait()

        def bbody(j, carry):
            accs = row_pairs(srows_v[s], j, 0.9)
            brow = jnp.full((LANES,), c * BC + j, jnp.int32)
            for k in range(K):
                swi = plsc.load_gather(
                    nidx_v, [brow, jnp.full((LANES,), K + k, jnp.int32)])
                sw = plsc.bitcast(swi, jnp.float32) * 0.1
                rows = row_pairs(nrows_v[s], j * K + k, sw)
                accs = tuple(acc + rv for acc, rv in zip(accs, rows))
            for g in range(2 * NG):
                out_v[s][j, pl.ds(g * LANES, LANES)] = accs[g]
            return carry

        lax.fori_loop(0, BC, bbody, 0)
        out_copy(c, s).start()

    issue(0, 0)

    def pair(i, carry):
        c0 = 2 * i
        issue(c0 + 1, 1)
        drain(c0, 0)
        compute(c0, 0)

        @pl.when(c0 + 2 < NCH)
        def _():
            issue(c0 + 2, 0)

        drain(c0 + 1, 1)
        compute(c0 + 1, 1)
        return carry

    lax.fori_loop(0, NCH // 2, pair, 0)
    out_copy(NCH - 2, 0).wait()
    out_copy(NCH - 1, 1).wait()


@functools.partial(jax.jit, static_argnames=())
def _run(combined, source_nodes, idxw):
    mesh = plsc.VectorSubcoreMesh(core_axis_name="c", subcore_axis_name="s")
    sc_kernel = pl.kernel(
        _sc_body,
        out_type=jax.ShapeDtypeStruct((B, D_FEAT), jnp.float32),
        mesh=mesh,
        compiler_params=pltpu.CompilerParams(needs_layout_passes=False),
        scratch_types=[
            pltpu.VMEM((B_PER_W, 2 * K), jnp.int32),
            pltpu.VMEM((B_PER_W,), jnp.int32),
            [pltpu.VMEM((BCK, DH), jnp.int32)] * 2,
            [pltpu.VMEM((BC, DH), jnp.int32)] * 2,
            [pltpu.VMEM((BC, D_FEAT), jnp.float32)] * 2,
            [pltpu.SemaphoreType.DMA] * 2,
            [pltpu.SemaphoreType.DMA] * 2,
        ],
    )
    return sc_kernel(combined, source_nodes, idxw)


def kernel(node_features, edge_features, memory, w_time, b_time, timestamps,
           edge_times, tppr_weights, source_nodes, neighbors, edge_idxs):
    combined = _combine(node_features, memory)
    idxw = jnp.concatenate(
        [neighbors.astype(jnp.int32),
         lax.bitcast_convert_type(tppr_weights, jnp.int32)], axis=1)
    return _run(combined, source_nodes.astype(jnp.int32), idxw)
